# Initial kernel scaffold; baseline (speedup 1.0000x reference)
import functools

import jax
import jax.numpy as jnp
import numpy as np
from jax import lax
from jax.experimental import pallas as pl
from jax.experimental.pallas import tpu as pltpu
from jax.experimental.pallas import tpu_sc as plsc

N_NODES = 10000
E = 320000
H = 8
DK = 16
D = H * DK
NL = 2
VOCAB = 1000
MAXPOS = 4096
DFF = 512

NC = 2
NS = 16
L = 16
NW = NC * NS

B = 80
ROWS_PER_SUB = N_NODES // NS


def _sc_mesh():
    return plsc.VectorSubcoreMesh(core_axis_name="c", subcore_axis_name="s")


def _embed(tok, pos, value_table, coord_table, pos_table):
    n_chunks = N_NODES // B
    per_worker = -(-n_chunks // NW)

    @functools.partial(
        pl.kernel,
        out_type=jax.ShapeDtypeStruct((N_NODES, D), jnp.float32),
        mesh=_sc_mesh(),
        scratch_types=[
            pltpu.VMEM((B,), jnp.int32),
            pltpu.VMEM((B,), jnp.int32),
            pltpu.VMEM((B,), jnp.int32),
            pltpu.VMEM((B,), jnp.int32),
            pltpu.VMEM((B, D), jnp.float32),
            pltpu.VMEM((B, D), jnp.float32),
            pltpu.VMEM((B, D), jnp.float32),
        ],
    )
    def k(tok_hbm, pos_hbm, vt_hbm, ct_hbm, pt_hbm, x_hbm,
          tok_v, pos_v, cidx, pidx, vbuf, cbuf, pbuf):
        cid = lax.axis_index("c")
        sid = lax.axis_index("s")
        wid = sid * NC + cid

        def chunk_body(c):
            base = c * B
            pltpu.sync_copy(tok_hbm.at[pl.ds(base, B)], tok_v)
            pltpu.sync_copy(pos_hbm.at[pl.ds(base, B)], pos_v)
            for i in range(B // L):
                sl = pl.ds(i * L, L)
                p = pos_v[sl]
                cidx[sl] = lax.rem(p, 3)
                pidx[sl] = lax.div(p, 3)
            pltpu.sync_copy(vt_hbm.at[tok_v], vbuf)
            pltpu.sync_copy(ct_hbm.at[cidx], cbuf)
            pltpu.sync_copy(pt_hbm.at[pidx], pbuf)

            def add_body(r, _):
                for j in range(D // L):
                    sl = pl.ds(j * L, L)
                    vbuf[r, sl] = vbuf[r, sl] + cbuf[r, sl] + pbuf[r, sl]
                return 0

            lax.fori_loop(0, B, add_body, 0)
            pltpu.sync_copy(vbuf, x_hbm.at[pl.ds(base, B)])

        for t in range(per_worker):
            c = wid + t * NW

            @pl.when(c < n_chunks)
            def _():
                chunk_body(c)

    return k(tok, pos, value_table, coord_table, pos_table)


def _edge(q, kv, edge_index):
    chunks_per_worker = E // NW // B

    @functools.partial(
        pl.kernel,
        out_type=(
            jax.ShapeDtypeStruct((NC, N_NODES, D), jnp.float32),
            jax.ShapeDtypeStruct((NC, N_NODES, L), jnp.float32),
        ),
        mesh=_sc_mesh(),
        scratch_types=[
            pltpu.VMEM_SHARED((N_NODES, D), jnp.float32),
            pltpu.VMEM_SHARED((N_NODES, L), jnp.float32),
            pltpu.VMEM((B,), jnp.int32),
            pltpu.VMEM((B,), jnp.int32),
            pltpu.VMEM((B, 2 * D), jnp.float32),
            pltpu.VMEM((B, D), jnp.float32),
            pltpu.VMEM((B, D), jnp.float32),
            pltpu.VMEM((B, L), jnp.float32),
        ],
    )
    def k(q_hbm, kv_hbm, ei_hbm, num_hbm, den_hbm,
          num_sp, den_sp, sidx, didx, kvbuf, qbuf, wbuf, denb):
        cid = lax.axis_index("c")
        sid = lax.axis_index("s")
        zero = jnp.zeros((L,), jnp.float32)
        lane = lax.iota(jnp.int32, L)

        def zb(r, _):
            for j in range(D // L):
                wbuf[r, pl.ds(j * L, L)] = zero
            denb[r, pl.ds(0, L)] = zero
            return 0

        lax.fori_loop(0, B, zb, 0)
        r0 = sid * ROWS_PER_SUB
        for i in range(-(-ROWS_PER_SUB // B)):
            rows = min(B, ROWS_PER_SUB - i * B)
            pltpu.sync_copy(wbuf.at[pl.ds(0, rows)],
                            num_sp.at[pl.ds(r0 + i * B, rows)])
            pltpu.sync_copy(denb.at[pl.ds(0, rows)],
                            den_sp.at[pl.ds(r0 + i * B, rows)])
        plsc.subcore_barrier()

        ebase = cid * (E // NC) + sid * (E // NC // NS)

        def chunk(t, _):
            base = ebase + t * B
            pltpu.sync_copy(ei_hbm.at[0, pl.ds(base, B)], sidx)
            pltpu.sync_copy(ei_hbm.at[1, pl.ds(base, B)], didx)
            pltpu.sync_copy(kv_hbm.at[sidx], kvbuf)
            pltpu.sync_copy(q_hbm.at[didx], qbuf)

            def edge_j(j, _):
                den_vec = zero
                for h in range(H):
                    sl = pl.ds(h * L, L)
                    kh = kvbuf[j, sl]
                    qh = qbuf[j, sl]
                    s = jnp.sum(kh * qh)
                    e_vec = jnp.exp(jnp.broadcast_to(s, (L,)))
                    vh = kvbuf[j, pl.ds(D + h * L, L)]
                    wbuf[j, sl] = vh * e_vec
                    den_vec = jnp.where(lane == h, e_vec, den_vec)
                denb[j, pl.ds(0, L)] = den_vec
                return 0

            lax.fori_loop(0, B, edge_j, 0)
            pltpu.sync_copy(wbuf, num_sp.at[didx], add=True)
            pltpu.sync_copy(denb, den_sp.at[didx], add=True)
            return 0

        lax.fori_loop(0, chunks_per_worker, chunk, 0)
        plsc.subcore_barrier()

        for i in range(-(-ROWS_PER_SUB // B)):
            rows = min(B, ROWS_PER_SUB - i * B)
            rr = r0 + i * B
            pltpu.sync_copy(num_sp.at[pl.ds(rr, rows)],
                            num_hbm.at[cid, pl.ds(rr, rows)])
            pltpu.sync_copy(den_sp.at[pl.ds(rr, rows)],
                            den_hbm.at[cid, pl.ds(rr, rows)])

    return k(q, kv, edge_index)


def _ln(x, eps=1e-5):
    mu = jnp.mean(x, axis=-1, keepdims=True)
    d = x - mu
    var = jnp.mean(d * d, axis=-1, keepdims=True)
    return d * lax.rsqrt(var + eps)


_RB = 1000


def _ln_qkv(x, wqkv):
    scale = 1.0 / np.sqrt(np.float32(DK))

    def body(x_ref, w_ref, q_ref, kv_ref):
        xn = _ln(x_ref[...])
        qkv = jnp.dot(xn, w_ref[...], preferred_element_type=jnp.float32)
        q_ref[...] = qkv[:, :D] * scale
        kv_ref[...] = qkv[:, D:]

    return pl.pallas_call(
        body,
        grid=(N_NODES // _RB,),
        in_specs=[
            pl.BlockSpec((_RB, D), lambda i: (i, 0)),
            pl.BlockSpec((D, 3 * D), lambda i: (0, 0)),
        ],
        out_specs=[
            pl.BlockSpec((_RB, D), lambda i: (i, 0)),
            pl.BlockSpec((_RB, 2 * D), lambda i: (i, 0)),
        ],
        out_shape=(
            jax.ShapeDtypeStruct((N_NODES, D), jnp.float32),
            jax.ShapeDtypeStruct((N_NODES, 2 * D), jnp.float32),
        ),
    )(x, wqkv)


def _post(x, num, den, wo, w1, w2):
    def body(x_ref, num_ref, den_ref, wo_ref, w1_ref, w2_ref, o_ref):
        xv = x_ref[...]
        numv = num_ref[0] + num_ref[1]
        denv = den_ref[0] + den_ref[1]
        den8 = denv[:, :H]
        row = lax.broadcasted_iota(jnp.int32, (H, D), 0)
        col = lax.broadcasted_iota(jnp.int32, (H, D), 1)
        em = (col // DK == row).astype(jnp.float32)
        den_exp = jnp.dot(den8, em, preferred_element_type=jnp.float32)
        z = numv / (den_exp + 1e-9)
        xv = xv + jnp.dot(z, wo_ref[...], preferred_element_type=jnp.float32)
        xn = _ln(xv)
        h1 = jnp.maximum(
            jnp.dot(xn, w1_ref[...], preferred_element_type=jnp.float32), 0.0)
        o_ref[...] = xv + jnp.dot(h1, w2_ref[...],
                                  preferred_element_type=jnp.float32)

    return pl.pallas_call(
        body,
        grid=(N_NODES // _RB,),
        in_specs=[
            pl.BlockSpec((_RB, D), lambda i: (i, 0)),
            pl.BlockSpec((NC, _RB, D), lambda i: (0, i, 0)),
            pl.BlockSpec((NC, _RB, L), lambda i: (0, i, 0)),
            pl.BlockSpec((D, D), lambda i: (0, 0)),
            pl.BlockSpec((D, DFF), lambda i: (0, 0)),
            pl.BlockSpec((DFF, D), lambda i: (0, 0)),
        ],
        out_specs=pl.BlockSpec((_RB, D), lambda i: (i, 0)),
        out_shape=jax.ShapeDtypeStruct((N_NODES, D), jnp.float32),
    )(x, num, den, wo, w1, w2)


def _generator(x, wgen):
    def body(x_ref, w_ref, o_ref):
        xn = _ln(x_ref[...])
        logits = jnp.dot(xn, w_ref[...], preferred_element_type=jnp.float32)
        m = jnp.max(logits, axis=-1, keepdims=True)
        s = logits - m
        o_ref[...] = s - jnp.log(jnp.sum(jnp.exp(s), axis=-1, keepdims=True))

    return pl.pallas_call(
        body,
        grid=(N_NODES // _RB,),
        in_specs=[
            pl.BlockSpec((_RB, D), lambda i: (i, 0)),
            pl.BlockSpec((D, VOCAB), lambda i: (0, 0)),
        ],
        out_specs=pl.BlockSpec((_RB, VOCAB), lambda i: (i, 0)),
        out_shape=jax.ShapeDtypeStruct((N_NODES, VOCAB), jnp.float32),
    )(x, wgen)


def kernel(tgt_tokens, tgt_pos, edge_index, value_table, coord_table,
           pos_table, Wqkv, Wo, W1, W2, Wgen):
    tok = tgt_tokens.astype(jnp.int32)
    pos = tgt_pos.astype(jnp.int32)
    ei = edge_index.astype(jnp.int32)
    x = _embed(tok, pos, value_table, coord_table, pos_table)
    for i in range(NL):
        q, kv = _ln_qkv(x, Wqkv[i])
        num, den = _edge(q, kv, ei)
        x = _post(x, num, den, Wo[i], W1[i], W2[i])
    return _generator(x, Wgen)


# speedup vs baseline: 11.6712x; 11.6712x over previous
import functools

import jax
import jax.numpy as jnp
import numpy as np
from jax import lax
from jax.experimental import pallas as pl
from jax.experimental.pallas import tpu as pltpu
from jax.experimental.pallas import tpu_sc as plsc

N_NODES = 10000
E = 320000
H = 8
DK = 16
D = H * DK
NL = 2
VOCAB = 1000
MAXPOS = 4096
DFF = 512

NC = 2
NS = 16
L = 16
NW = NC * NS

B = 80
ROWS_PER_SUB = N_NODES // NS


def _sc_mesh():
    return plsc.VectorSubcoreMesh(core_axis_name="c", subcore_axis_name="s")


_SC_PARAMS = pltpu.CompilerParams(needs_layout_passes=False,
                                  use_tc_tiling_on_sc=False)


def _embed(tok, pos, value_table, coord_table, pos_table):
    n_chunks = N_NODES // B
    per_worker = -(-n_chunks // NW)

    @functools.partial(
        pl.kernel,
        out_type=jax.ShapeDtypeStruct((N_NODES, D), jnp.float32),
        mesh=_sc_mesh(),
        compiler_params=_SC_PARAMS,
        scratch_types=[
            pltpu.VMEM((B,), jnp.int32),
            pltpu.VMEM((B,), jnp.int32),
            pltpu.VMEM((B,), jnp.int32),
            pltpu.VMEM((B,), jnp.int32),
            pltpu.VMEM((B, D), jnp.float32),
            pltpu.VMEM((B, D), jnp.float32),
            pltpu.VMEM((B, D), jnp.float32),
        ],
    )
    def k(tok_hbm, pos_hbm, vt_hbm, ct_hbm, pt_hbm, x_hbm,
          tok_v, pos_v, cidx, pidx, vbuf, cbuf, pbuf):
        cid = lax.axis_index("c")
        sid = lax.axis_index("s")
        wid = sid * NC + cid

        def chunk_body(c):
            base = c * B
            pltpu.sync_copy(tok_hbm.at[pl.ds(base, B)], tok_v)
            pltpu.sync_copy(pos_hbm.at[pl.ds(base, B)], pos_v)
            for i in range(B // L):
                sl = pl.ds(i * L, L)
                p = pos_v[sl]
                cidx[sl] = lax.rem(p, 3)
                pidx[sl] = lax.div(p, 3)
            pltpu.sync_copy(vt_hbm.at[tok_v], vbuf)
            pltpu.sync_copy(ct_hbm.at[cidx], cbuf)
            pltpu.sync_copy(pt_hbm.at[pidx], pbuf)

            def add_body(r, _):
                for j in range(D // L):
                    sl = pl.ds(j * L, L)
                    vbuf[r, sl] = vbuf[r, sl] + cbuf[r, sl] + pbuf[r, sl]
                return 0

            lax.fori_loop(0, B, add_body, 0)
            pltpu.sync_copy(vbuf, x_hbm.at[pl.ds(base, B)])

        for t in range(per_worker):
            c = wid + t * NW

            @pl.when(c < n_chunks)
            def _():
                chunk_body(c)

    return k(tok, pos, value_table, coord_table, pos_table)


EB = 48
E_PAD = 320016
N_ACC = 10032


def _edge(q, kv, src, dst):
    n_edge_chunks = E_PAD // EB

    @functools.partial(
        pl.kernel,
        out_type=(
            jax.ShapeDtypeStruct((NC, N_NODES, D), jnp.float32),
            jax.ShapeDtypeStruct((NC, N_NODES, L), jnp.float32),
        ),
        mesh=_sc_mesh(),
        compiler_params=_SC_PARAMS,
        scratch_types=[
            pltpu.VMEM_SHARED((N_ACC, D), jnp.float32),
            pltpu.VMEM_SHARED((N_ACC, L), jnp.float32),
            pltpu.VMEM((EB,), jnp.int32),
            pltpu.VMEM((EB,), jnp.int32),
            pltpu.VMEM((EB, 2 * D), jnp.float32),
            pltpu.VMEM((EB, D), jnp.float32),
            pltpu.VMEM((EB, D), jnp.float32),
            pltpu.VMEM((EB, L), jnp.float32),
        ],
    )
    def k(q_hbm, kv_hbm, src_hbm, dst_hbm, num_hbm, den_hbm,
          num_sp, den_sp, sidx, didx, kvbuf, qbuf, wbuf, denb):
        cid = lax.axis_index("c")
        sid = lax.axis_index("s")
        wid = sid * NC + cid
        zero = jnp.zeros((L,), jnp.float32)
        lane = lax.iota(jnp.int32, L)

        def zb(r, _):
            for j in range(D // L):
                wbuf[r, pl.ds(j * L, L)] = zero
            denb[r, pl.ds(0, L)] = zero
            return 0

        lax.fori_loop(0, EB, zb, 0)
        n_zero_chunks = N_ACC // EB
        for t in range(-(-n_zero_chunks // NS)):
            c = sid + t * NS

            @pl.when(c < n_zero_chunks)
            def _():
                pltpu.sync_copy(wbuf, num_sp.at[pl.ds(c * EB, EB)])
                pltpu.sync_copy(denb, den_sp.at[pl.ds(c * EB, EB)])
        plsc.subcore_barrier()

        def chunk(cidx_, _):
            base = cidx_ * EB
            pltpu.sync_copy(src_hbm.at[pl.ds(base, EB)], sidx)
            pltpu.sync_copy(dst_hbm.at[pl.ds(base, EB)], didx)
            pltpu.sync_copy(kv_hbm.at[sidx], kvbuf)
            pltpu.sync_copy(q_hbm.at[didx], qbuf)

            def edge_g(g, _):
                rows = lane + g * L
                for h in range(H):
                    acc = zero
                    for d in range(DK):
                        col = jnp.full((L,), h * DK + d, jnp.int32)
                        kg = plsc.load_gather(kvbuf, [rows, col])
                        qg = plsc.load_gather(qbuf, [rows, col])
                        acc = acc + kg * qg
                    e_vec = jnp.exp(acc)
                    plsc.store_scatter(
                        denb, [rows, jnp.full((L,), h, jnp.int32)], e_vec)
                    for d in range(DK):
                        cv = jnp.full((L,), D + h * DK + d, jnp.int32)
                        cw = jnp.full((L,), h * DK + d, jnp.int32)
                        vg = plsc.load_gather(kvbuf, [rows, cv])
                        plsc.store_scatter(wbuf, [rows, cw], vg * e_vec)
                return 0

            lax.fori_loop(0, EB // L, edge_g, 0)
            pltpu.sync_copy(wbuf, num_sp.at[didx], add=True)
            pltpu.sync_copy(denb, den_sp.at[didx], add=True)
            return 0

        def chunk_loop(t, _):
            c = wid + t * NW

            @pl.when(c < n_edge_chunks)
            def _():
                chunk(c, None)
            return 0

        lax.fori_loop(0, -(-n_edge_chunks // NW), chunk_loop, 0)
        plsc.subcore_barrier()

        WB = 40
        n_wb_chunks = N_NODES // WB
        for t in range(-(-n_wb_chunks // NS)):
            c = sid + t * NS

            @pl.when(c < n_wb_chunks)
            def _():
                pltpu.sync_copy(num_sp.at[pl.ds(c * WB, WB)],
                                num_hbm.at[cid, pl.ds(c * WB, WB)])
                pltpu.sync_copy(den_sp.at[pl.ds(c * WB, WB)],
                                den_hbm.at[cid, pl.ds(c * WB, WB)])

    return k(q, kv, src, dst)


def _ln(x, eps=1e-5):
    mu = jnp.mean(x, axis=-1, keepdims=True)
    d = x - mu
    var = jnp.mean(d * d, axis=-1, keepdims=True)
    return d * lax.rsqrt(var + eps)


_RB = 1000


def _ln_qkv(x, wqkv):
    scale = 1.0 / np.sqrt(np.float32(DK))

    def body(x_ref, w_ref, q_ref, kv_ref):
        xn = _ln(x_ref[...])
        qkv = jnp.dot(xn, w_ref[...], preferred_element_type=jnp.float32)
        q_ref[...] = qkv[:, :D] * scale
        kv_ref[...] = qkv[:, D:]

    return pl.pallas_call(
        body,
        grid=(N_NODES // _RB,),
        in_specs=[
            pl.BlockSpec((_RB, D), lambda i: (i, 0)),
            pl.BlockSpec((D, 3 * D), lambda i: (0, 0)),
        ],
        out_specs=[
            pl.BlockSpec((_RB, D), lambda i: (i, 0)),
            pl.BlockSpec((_RB, 2 * D), lambda i: (i, 0)),
        ],
        out_shape=(
            jax.ShapeDtypeStruct((N_NODES, D), jnp.float32),
            jax.ShapeDtypeStruct((N_NODES, 2 * D), jnp.float32),
        ),
    )(x, wqkv)


def _post(x, num, den, wo, w1, w2):
    def body(x_ref, num_ref, den_ref, wo_ref, w1_ref, w2_ref, o_ref):
        xv = x_ref[...]
        numv = num_ref[0] + num_ref[1]
        denv = den_ref[0] + den_ref[1]
        den8 = denv[:, :H]
        row = lax.broadcasted_iota(jnp.int32, (H, D), 0)
        col = lax.broadcasted_iota(jnp.int32, (H, D), 1)
        em = (col // DK == row).astype(jnp.float32)
        den_exp = jnp.dot(den8, em, preferred_element_type=jnp.float32)
        z = numv / (den_exp + 1e-9)
        xv = xv + jnp.dot(z, wo_ref[...], preferred_element_type=jnp.float32)
        xn = _ln(xv)
        h1 = jnp.maximum(
            jnp.dot(xn, w1_ref[...], preferred_element_type=jnp.float32), 0.0)
        o_ref[...] = xv + jnp.dot(h1, w2_ref[...],
                                  preferred_element_type=jnp.float32)

    return pl.pallas_call(
        body,
        grid=(N_NODES // _RB,),
        in_specs=[
            pl.BlockSpec((_RB, D), lambda i: (i, 0)),
            pl.BlockSpec((NC, _RB, D), lambda i: (0, i, 0)),
            pl.BlockSpec((NC, _RB, L), lambda i: (0, i, 0)),
            pl.BlockSpec((D, D), lambda i: (0, 0)),
            pl.BlockSpec((D, DFF), lambda i: (0, 0)),
            pl.BlockSpec((DFF, D), lambda i: (0, 0)),
        ],
        out_specs=pl.BlockSpec((_RB, D), lambda i: (i, 0)),
        out_shape=jax.ShapeDtypeStruct((N_NODES, D), jnp.float32),
    )(x, num, den, wo, w1, w2)


def _generator(x, wgen):
    def body(x_ref, w_ref, o_ref):
        xn = _ln(x_ref[...])
        logits = jnp.dot(xn, w_ref[...], preferred_element_type=jnp.float32)
        m = jnp.max(logits, axis=-1, keepdims=True)
        s = logits - m
        o_ref[...] = s - jnp.log(jnp.sum(jnp.exp(s), axis=-1, keepdims=True))

    return pl.pallas_call(
        body,
        grid=(N_NODES // _RB,),
        in_specs=[
            pl.BlockSpec((_RB, D), lambda i: (i, 0)),
            pl.BlockSpec((D, VOCAB), lambda i: (0, 0)),
        ],
        out_specs=pl.BlockSpec((_RB, VOCAB), lambda i: (i, 0)),
        out_shape=jax.ShapeDtypeStruct((N_NODES, VOCAB), jnp.float32),
    )(x, wgen)


def kernel(tgt_tokens, tgt_pos, edge_index, value_table, coord_table,
           pos_table, Wqkv, Wo, W1, W2, Wgen):
    tok = tgt_tokens.astype(jnp.int32)
    pos = tgt_pos.astype(jnp.int32)
    src = jnp.concatenate(
        [edge_index[0].astype(jnp.int32),
         jnp.zeros((E_PAD - E,), jnp.int32)])
    dst = jnp.concatenate(
        [edge_index[1].astype(jnp.int32),
         jnp.full((E_PAD - E,), N_NODES, jnp.int32)])
    x = _embed(tok, pos, value_table, coord_table, pos_table)
    for i in range(NL):
        q, kv = _ln_qkv(x, Wqkv[i])
        num, den = _edge(q, kv, src, dst)
        x = _post(x, num, den, Wo[i], W1[i], W2[i])
    return _generator(x, Wgen)


# speedup vs baseline: 14.6311x; 1.2536x over previous
import functools

import jax
import jax.numpy as jnp
import numpy as np
from jax import lax
from jax.experimental import pallas as pl
from jax.experimental.pallas import tpu as pltpu
from jax.experimental.pallas import tpu_sc as plsc

N_NODES = 10000
E = 320000
H = 8
DK = 16
D = H * DK
NL = 2
VOCAB = 1000
MAXPOS = 4096
DFF = 512

NC = 2
NS = 16
L = 16
NW = NC * NS

AW = D + H

EB = 32
CPW = 314
E_PAD = EB * CPW * NW
N_ACC = EB * CPW


def _sc_mesh():
    return plsc.VectorSubcoreMesh(core_axis_name="c", subcore_axis_name="s")


_SC_PARAMS = pltpu.CompilerParams(needs_layout_passes=False,
                                  use_tc_tiling_on_sc=False)


def _embed(tok, pos, value_table, coord_table, pos_table):
    B = 80
    n_chunks = N_NODES // B
    per_worker = -(-n_chunks // NW)

    @functools.partial(
        pl.kernel,
        out_type=jax.ShapeDtypeStruct((N_NODES, D), jnp.float32),
        mesh=_sc_mesh(),
        compiler_params=_SC_PARAMS,
        scratch_types=[
            pltpu.VMEM((B,), jnp.int32),
            pltpu.VMEM((B,), jnp.int32),
            pltpu.VMEM((B,), jnp.int32),
            pltpu.VMEM((B,), jnp.int32),
            pltpu.VMEM((B, D), jnp.float32),
            pltpu.VMEM((B, D), jnp.float32),
            pltpu.VMEM((B, D), jnp.float32),
        ],
    )
    def k(tok_hbm, pos_hbm, vt_hbm, ct_hbm, pt_hbm, x_hbm,
          tok_v, pos_v, cidx, pidx, vbuf, cbuf, pbuf):
        cid = lax.axis_index("c")
        sid = lax.axis_index("s")
        wid = sid * NC + cid

        def chunk_body(c):
            base = c * B
            pltpu.sync_copy(tok_hbm.at[pl.ds(base, B)], tok_v)
            pltpu.sync_copy(pos_hbm.at[pl.ds(base, B)], pos_v)
            for i in range(B // L):
                sl = pl.ds(i * L, L)
                p = pos_v[sl]
                cidx[sl] = lax.rem(p, 3)
                pidx[sl] = lax.div(p, 3)
            pltpu.sync_copy(vt_hbm.at[tok_v], vbuf)
            pltpu.sync_copy(ct_hbm.at[cidx], cbuf)
            pltpu.sync_copy(pt_hbm.at[pidx], pbuf)

            def add_body(r, _):
                for j in range(D // L):
                    sl = pl.ds(j * L, L)
                    vbuf[r, sl] = vbuf[r, sl] + cbuf[r, sl] + pbuf[r, sl]
                return 0

            lax.fori_loop(0, B, add_body, 0)
            pltpu.sync_copy(vbuf, x_hbm.at[pl.ds(base, B)])

        for t in range(per_worker):
            c = wid + t * NW

            @pl.when(c < n_chunks)
            def _():
                chunk_body(c)

    return k(tok, pos, value_table, coord_table, pos_table)


def _edge(q, kv, src, dst):
    @functools.partial(
        pl.kernel,
        out_type=jax.ShapeDtypeStruct((NC, N_NODES, AW), jnp.float32),
        mesh=_sc_mesh(),
        compiler_params=_SC_PARAMS,
        scratch_types=[
            pltpu.VMEM_SHARED((N_ACC, AW), jnp.float32),
            pltpu.VMEM((EB,), jnp.int32),
            pltpu.VMEM((EB,), jnp.int32),
            pltpu.VMEM((EB,), jnp.int32),
            pltpu.VMEM((EB,), jnp.int32),
            pltpu.VMEM((EB,), jnp.int32),
            pltpu.VMEM((EB, 2 * D), jnp.float32),
            pltpu.VMEM((EB, 2 * D), jnp.float32),
            pltpu.VMEM((EB, D), jnp.float32),
            pltpu.VMEM((EB, D), jnp.float32),
            pltpu.VMEM((EB, AW), jnp.float32),
            pltpu.SemaphoreType.DMA,
            pltpu.SemaphoreType.DMA,
            pltpu.SemaphoreType.DMA,
        ],
    )
    def k(q_hbm, kv_hbm, src_hbm, dst_hbm, acc_hbm,
          acc_sp, si0, si1, di0, di1, dsc, kv0, kv1, q0, q1, w,
          gsem0, gsem1, ssem):
        cid = lax.axis_index("c")
        sid = lax.axis_index("s")
        wid = sid * NC + cid
        zero = jnp.zeros((L,), jnp.float32)
        lane = lax.iota(jnp.int32, L)
        si = (si0, si1)
        di = (di0, di1)
        kvb = (kv0, kv1)
        qb = (q0, q1)
        gsem = (gsem0, gsem1)

        def zb(r, _):
            for j in range(AW // L):
                w[r, pl.ds(j * L, L)] = zero
            w[r, pl.ds(AW - L, L)] = zero
            return 0

        lax.fori_loop(0, EB, zb, 0)
        for t in range(-(-CPW // NS)):
            c = sid + t * NS

            @pl.when(c < CPW)
            def _():
                pltpu.sync_copy(w, acc_sp.at[pl.ds(c * EB, EB)])
        plsc.subcore_barrier()

        def issue_gather(t, sl):
            base = (wid + t * NW) * EB
            pltpu.sync_copy(src_hbm.at[pl.ds(base, EB)], si[sl])
            pltpu.sync_copy(dst_hbm.at[pl.ds(base, EB)], di[sl])
            pltpu.async_copy(kv_hbm.at[si[sl]], kvb[sl], gsem[sl])
            pltpu.async_copy(q_hbm.at[di[sl]], qb[sl], gsem[sl])

        def compute(sl):
            kvr = kvb[sl]
            qr = qb[sl]
            for g in range(EB // L):
                rows = lane + g * L
                for h in range(H):
                    acc = zero
                    for d in range(DK):
                        col = jnp.full((L,), h * DK + d, jnp.int32)
                        kg = plsc.load_gather(kvr, [rows, col])
                        qg = plsc.load_gather(qr, [rows, col])
                        acc = acc + kg * qg
                    e_vec = jnp.exp(acc)
                    plsc.store_scatter(
                        w, [rows, jnp.full((L,), D + h, jnp.int32)], e_vec)
                    for d in range(DK):
                        cv = jnp.full((L,), D + h * DK + d, jnp.int32)
                        cw = jnp.full((L,), h * DK + d, jnp.int32)
                        vg = plsc.load_gather(kvr, [rows, cv])
                        plsc.store_scatter(w, [rows, cw], vg * e_vec)

        issue_gather(0, 0)
        issue_gather(1, 1)

        def pair(i, _):
            for sl in range(2):
                t = 2 * i + sl
                pltpu.make_async_copy(kv_hbm.at[si[sl]], kvb[sl],
                                      gsem[sl]).wait()
                pltpu.make_async_copy(q_hbm.at[di[sl]], qb[sl],
                                      gsem[sl]).wait()
                @pl.when(t > 0)
                def _():
                    pltpu.make_async_copy(w, acc_sp.at[dsc], ssem).wait()

                compute(sl)
                for j in range(EB // L):
                    s2 = pl.ds(j * L, L)
                    dsc[s2] = di[sl][s2]
                pltpu.async_copy(w, acc_sp.at[dsc], ssem, add=True)

                @pl.when(t + 2 < CPW)
                def _():
                    issue_gather(t + 2, sl)
            return 0

        lax.fori_loop(0, CPW // 2, pair, 0)
        pltpu.make_async_copy(w, acc_sp.at[dsc], ssem).wait()
        plsc.subcore_barrier()

        WB = 40
        n_wb_chunks = N_NODES // WB
        for t in range(-(-n_wb_chunks // NS)):
            c = sid + t * NS

            @pl.when(c < n_wb_chunks)
            def _():
                pltpu.sync_copy(acc_sp.at[pl.ds(c * WB, WB)],
                                acc_hbm.at[cid, pl.ds(c * WB, WB)])

    return k(q, kv, src, dst)


def _ln(x, eps=1e-5):
    mu = jnp.mean(x, axis=-1, keepdims=True)
    d = x - mu
    var = jnp.mean(d * d, axis=-1, keepdims=True)
    return d * lax.rsqrt(var + eps)


_RB = 1000


def _ln_qkv(x, wqkv):
    scale = 1.0 / np.sqrt(np.float32(DK))

    def body(x_ref, w_ref, q_ref, kv_ref):
        xn = _ln(x_ref[...])
        qkv = jnp.dot(xn, w_ref[...], preferred_element_type=jnp.float32)
        q_ref[...] = qkv[:, :D] * scale
        kv_ref[...] = qkv[:, D:]

    return pl.pallas_call(
        body,
        grid=(N_NODES // _RB,),
        in_specs=[
            pl.BlockSpec((_RB, D), lambda i: (i, 0)),
            pl.BlockSpec((D, 3 * D), lambda i: (0, 0)),
        ],
        out_specs=[
            pl.BlockSpec((_RB, D), lambda i: (i, 0)),
            pl.BlockSpec((_RB, 2 * D), lambda i: (i, 0)),
        ],
        out_shape=(
            jax.ShapeDtypeStruct((N_NODES, D), jnp.float32),
            jax.ShapeDtypeStruct((N_NODES, 2 * D), jnp.float32),
        ),
    )(x, wqkv)


def _post(x, acc, wo, w1, w2):
    def body(x_ref, acc_ref, wo_ref, w1_ref, w2_ref, o_ref):
        xv = x_ref[...]
        accv = acc_ref[0] + acc_ref[1]
        numv = accv[:, :D]
        den8 = accv[:, D:]
        row = lax.broadcasted_iota(jnp.int32, (H, D), 0)
        col = lax.broadcasted_iota(jnp.int32, (H, D), 1)
        em = (col // DK == row).astype(jnp.float32)
        den_exp = jnp.dot(den8, em, preferred_element_type=jnp.float32)
        z = numv / (den_exp + 1e-9)
        xv = xv + jnp.dot(z, wo_ref[...], preferred_element_type=jnp.float32)
        xn = _ln(xv)
        h1 = jnp.maximum(
            jnp.dot(xn, w1_ref[...], preferred_element_type=jnp.float32), 0.0)
        o_ref[...] = xv + jnp.dot(h1, w2_ref[...],
                                  preferred_element_type=jnp.float32)

    return pl.pallas_call(
        body,
        grid=(N_NODES // _RB,),
        in_specs=[
            pl.BlockSpec((_RB, D), lambda i: (i, 0)),
            pl.BlockSpec((NC, _RB, AW), lambda i: (0, i, 0)),
            pl.BlockSpec((D, D), lambda i: (0, 0)),
            pl.BlockSpec((D, DFF), lambda i: (0, 0)),
            pl.BlockSpec((DFF, D), lambda i: (0, 0)),
        ],
        out_specs=pl.BlockSpec((_RB, D), lambda i: (i, 0)),
        out_shape=jax.ShapeDtypeStruct((N_NODES, D), jnp.float32),
    )(x, acc, wo, w1, w2)


def _generator(x, wgen):
    def body(x_ref, w_ref, o_ref):
        xn = _ln(x_ref[...])
        logits = jnp.dot(xn, w_ref[...], preferred_element_type=jnp.float32)
        m = jnp.max(logits, axis=-1, keepdims=True)
        s = logits - m
        o_ref[...] = s - jnp.log(jnp.sum(jnp.exp(s), axis=-1, keepdims=True))

    return pl.pallas_call(
        body,
        grid=(N_NODES // _RB,),
        in_specs=[
            pl.BlockSpec((_RB, D), lambda i: (i, 0)),
            pl.BlockSpec((D, VOCAB), lambda i: (0, 0)),
        ],
        out_specs=pl.BlockSpec((_RB, VOCAB), lambda i: (i, 0)),
        out_shape=jax.ShapeDtypeStruct((N_NODES, VOCAB), jnp.float32),
    )(x, wgen)


def kernel(tgt_tokens, tgt_pos, edge_index, value_table, coord_table,
           pos_table, Wqkv, Wo, W1, W2, Wgen):
    tok = tgt_tokens.astype(jnp.int32)
    pos = tgt_pos.astype(jnp.int32)
    src = jnp.concatenate(
        [edge_index[0].astype(jnp.int32),
         jnp.zeros((E_PAD - E,), jnp.int32)])
    dst = jnp.concatenate(
        [edge_index[1].astype(jnp.int32),
         jnp.full((E_PAD - E,), N_NODES, jnp.int32)])
    x = _embed(tok, pos, value_table, coord_table, pos_table)
    for i in range(NL):
        q, kv = _ln_qkv(x, Wqkv[i])
        acc = _edge(q, kv, src, dst)
        x = _post(x, acc, Wo[i], W1[i], W2[i])
    return _generator(x, Wgen)
